# baseline (speedup 1.0000x reference)
import dataclasses
import functools

import jax
import jax.numpy as jnp
from jax.experimental import pallas as pl
from jax.experimental.pallas import tpu as pltpu
from jax.experimental.pallas import tpu_sc as plsc

N = 10000
E = 320000
D = 128
NP = 10240

NUM_CORES = 2
NUM_SUBCORES = 16
NUM_WORKERS = NUM_CORES * NUM_SUBCORES
PER_WORKER = E // NUM_WORKERS
ROWS_PER_SUB = NP // NUM_SUBCORES


CHUNK = 128
EPW = 10240
NCH = EPW // CHUNK
E_PAD = EPW * NUM_WORKERS


@functools.partial(
    pl.kernel,
    out_type=jax.ShapeDtypeStruct((NUM_CORES, NP, D), jnp.float32),
    mesh=plsc.VectorSubcoreMesh(core_axis_name="c", subcore_axis_name="s"),
    scratch_types=[
        pltpu.VMEM_SHARED((NP, D), jnp.float32),
        pltpu.VMEM((NCH, CHUNK), jnp.int32),
        pltpu.VMEM((4, CHUNK), jnp.int32),
        pltpu.VMEM((CHUNK, D), jnp.float32),
        pltpu.VMEM((CHUNK, D), jnp.float32),
        pltpu.SemaphoreType.DMA,
        pltpu.SemaphoreType.DMA,
        pltpu.SemaphoreType.DMA,
        pltpu.SemaphoreType.DMA,
        pltpu.SemaphoreType.DMA((4,)),
    ],
    name="gcn_edge_agg",
)
def _agg_feat(table2, init1, src_h, dst_h, out, acc, dst2d,
              sidx4, msg_a, msg_b, sg_a, sg_b, ss_a, ss_b, si4):
  core = jax.lax.axis_index("c")
  sub = jax.lax.axis_index("s")
  wid = core * NUM_SUBCORES + sub
  rbase = sub * ROWS_PER_SUB
  rows = pl.ds(rbase, ROWS_PER_SUB)
  ebase = wid * EPW

  table = table2.at[core]

  @pl.when(core == 0)
  def _():
    pltpu.async_copy(table2.at[0].at[rows], acc.at[rows], ss_a)

  @pl.when(core == 1)
  def _():
    pltpu.async_copy(init1.at[rows], acc.at[rows], ss_a)

  pltpu.sync_copy(dst_h.at[wid], dst2d)

  def src_slice(c):
    return src_h.at[pl.ds(ebase + c * CHUNK, CHUNK)]

  def slot(c):
    return jax.lax.rem(c, 4) if not isinstance(c, int) else c % 4

  def start_idx(c):
    pltpu.async_copy(src_slice(c), sidx4.at[slot(c)], si4.at[slot(c)])

  def wait_idx(c):
    pltpu.make_async_copy(src_slice(c), sidx4.at[slot(c)],
                          si4.at[slot(c)]).wait()

  def start_gather(c, msg, sem):
    pltpu.async_copy(table.at[sidx4.at[slot(c)]], msg, sem)

  def wait_gather(c, msg, sem):
    pltpu.make_async_copy(table.at[sidx4.at[slot(c)]], msg, sem).wait()

  def start_scatter(c, msg, sem):
    pltpu.async_copy(msg, acc.at[dst2d.at[c]], sem, add=True)

  def wait_scatter(c, msg, sem):
    pltpu.make_async_copy(msg, acc.at[dst2d.at[c]], sem).wait()

  pltpu.sync_copy(src_slice(0), sidx4.at[0])
  pltpu.sync_copy(src_slice(1), sidx4.at[1])
  start_gather(0, msg_a, sg_a)
  start_gather(1, msg_b, sg_b)
  start_idx(2)
  start_idx(3)

  @pl.when(core == 0)
  def _():
    pltpu.make_async_copy(table2.at[0].at[rows], acc.at[rows], ss_a).wait()

  @pl.when(core == 1)
  def _():
    pltpu.make_async_copy(init1.at[rows], acc.at[rows], ss_a).wait()

  plsc.subcore_barrier()
  wait_gather(0, msg_a, sg_a)
  start_scatter(0, msg_a, ss_a)

  @pl.loop(1, NCH // 2 - 1)
  def _(j):
    c0 = 2 * j
    wait_scatter(c0 - 2, msg_a, ss_a)
    start_idx(c0 + 2)
    wait_idx(c0)
    start_gather(c0, msg_a, sg_a)
    wait_gather(c0 - 1, msg_b, sg_b)
    start_scatter(c0 - 1, msg_b, ss_b)
    wait_scatter(c0 - 1, msg_b, ss_b)
    start_idx(c0 + 3)
    wait_idx(c0 + 1)
    start_gather(c0 + 1, msg_b, sg_b)
    wait_gather(c0, msg_a, sg_a)
    start_scatter(c0, msg_a, ss_a)

  wait_scatter(NCH - 4, msg_a, ss_a)
  wait_idx(NCH - 2)
  start_gather(NCH - 2, msg_a, sg_a)
  wait_gather(NCH - 3, msg_b, sg_b)
  start_scatter(NCH - 3, msg_b, ss_b)
  wait_scatter(NCH - 3, msg_b, ss_b)
  wait_idx(NCH - 1)
  start_gather(NCH - 1, msg_b, sg_b)
  wait_gather(NCH - 2, msg_a, sg_a)
  start_scatter(NCH - 2, msg_a, ss_a)
  wait_gather(NCH - 1, msg_b, sg_b)
  start_scatter(NCH - 1, msg_b, ss_b)
  wait_scatter(NCH - 2, msg_a, ss_a)
  wait_scatter(NCH - 1, msg_b, ss_b)

  plsc.subcore_barrier()
  pltpu.sync_copy(acc.at[rows], out.at[core].at[rows])

_DEG_CHUNK = 2000

_sc_params = pltpu.CompilerParams()
if "needs_layout_passes" in pltpu.CompilerParams.__dataclass_fields__:
  _sc_params = dataclasses.replace(_sc_params, needs_layout_passes=False)


@functools.partial(
    pl.kernel,
    out_type=jax.ShapeDtypeStruct((NUM_WORKERS, NP), jnp.float32),
    mesh=plsc.VectorSubcoreMesh(core_axis_name="c", subcore_axis_name="s"),
    scratch_types=[
        pltpu.VMEM((NP,), jnp.float32),
        pltpu.VMEM((_DEG_CHUNK,), jnp.int32),
    ],
    compiler_params=_sc_params,
    name="gcn_degree",
)
def _deg_kernel(dst_h, out, hist, didx):
  core = jax.lax.axis_index("c")
  sub = jax.lax.axis_index("s")
  wid = core * NUM_SUBCORES + sub
  zeros16 = jnp.zeros((16,), jnp.float32)
  ones16 = jnp.ones((16,), jnp.float32)

  @pl.loop(0, NP // 16)
  def _(i):
    hist[pl.ds(i * 16, 16)] = zeros16

  ebase = wid * PER_WORKER

  @pl.loop(0, PER_WORKER // _DEG_CHUNK)
  def _(i):
    pltpu.sync_copy(dst_h.at[pl.ds(ebase + i * _DEG_CHUNK, _DEG_CHUNK)], didx)

    @pl.loop(0, _DEG_CHUNK // 16)
    def _(j):
      idx = didx[pl.ds(j * 16, 16)]
      plsc.addupdate_scatter(hist, [idx], ones16)

  pltpu.sync_copy(hist, out.at[wid])


def _p1a_body(x_ref, w1_ref, h_ref):
  h_ref[...] = jax.lax.dot_general(x_ref[...], w1_ref[...],
                                   (((1,), (1,)), ((), ())),
                                   preferred_element_type=jnp.float32)


def _p1_body(degp_ref, h_ref, y1_ref, dis_ref):
  ones_col = jnp.ones((NUM_WORKERS, 1), jnp.float32)
  deg = 1.0 + jax.lax.dot_general(degp_ref[...], ones_col,
                                  (((0,), (0,)), ((), ())),
                                  preferred_element_type=jnp.float32)
  dis = jax.lax.rsqrt(deg)
  y = dis * h_ref[...]
  y1_ref[0] = y
  y1_ref[1] = y
  dis_ref[...] = dis


def _p2_body(p_ref, dis_ref, b1_ref, w2_ref, y2_ref):
  s = dis_ref[...] * (p_ref[0] + p_ref[1]) + b1_ref[...]
  h1r = jnp.maximum(s, 0.0)
  h2 = jax.lax.dot_general(h1r, w2_ref[...], (((1,), (1,)), ((), ())),
                           preferred_element_type=jnp.float32)
  y = dis_ref[...] * h2
  y2_ref[0] = y
  y2_ref[1] = y


def _p3_body(q_ref, dis_ref, b2_ref, x_ref, o_ref):
  o_ref[...] = dis_ref[...] * (q_ref[0] + q_ref[1]) + b2_ref[...] + x_ref[...]


def kernel(x, edge_index, W1, b1, W2, b2):
  src = edge_index[0]
  dst = edge_index[1]
  xp = jnp.pad(x, ((0, NP - N), (0, 0)))
  zeros_feat = jnp.zeros((NP, D), jnp.float32)

  pad_rows = N + (jnp.arange(E_PAD - E, dtype=jnp.int32) % (NP - N))
  srcp = jnp.concatenate([src, pad_rows])
  dstp = jnp.concatenate([dst, pad_rows]).reshape(NUM_WORKERS, NCH, CHUNK)

  degp = _deg_kernel(dst)
  h1 = pl.pallas_call(
      _p1a_body,
      out_shape=jax.ShapeDtypeStruct((NP, D), jnp.float32),
  )(xp, W1)

  y1, dis = pl.pallas_call(
      _p1_body,
      out_shape=(jax.ShapeDtypeStruct((2, NP, D), jnp.float32),
                 jax.ShapeDtypeStruct((NP, 1), jnp.float32)),
  )(degp, h1)

  p = _agg_feat(y1, zeros_feat, srcp, dstp)

  y2 = pl.pallas_call(
      _p2_body,
      out_shape=jax.ShapeDtypeStruct((2, NP, D), jnp.float32),
  )(p, dis, b1.reshape(1, D), W2)

  q = _agg_feat(y2, zeros_feat, srcp, dstp)

  out = pl.pallas_call(
      _p3_body,
      out_shape=jax.ShapeDtypeStruct((NP, D), jnp.float32),
  )(q, dis, b2.reshape(1, D), xp)

  return out[:N]


# speedup vs baseline: 1.0102x; 1.0102x over previous
import dataclasses
import functools

import jax
import jax.numpy as jnp
from jax.experimental import pallas as pl
from jax.experimental.pallas import tpu as pltpu
from jax.experimental.pallas import tpu_sc as plsc

N = 10000
E = 320000
D = 128
NP = 10240

NUM_CORES = 2
NUM_SUBCORES = 16
NUM_WORKERS = NUM_CORES * NUM_SUBCORES
PER_WORKER = E // NUM_WORKERS
ROWS_PER_SUB = NP // NUM_SUBCORES


CHUNK = 128
EPW = 10240
NCH = EPW // CHUNK
E_PAD = EPW * NUM_WORKERS


@functools.partial(
    pl.kernel,
    out_type=jax.ShapeDtypeStruct((NUM_CORES, NP, D), jnp.float32),
    mesh=plsc.VectorSubcoreMesh(core_axis_name="c", subcore_axis_name="s"),
    scratch_types=[
        pltpu.VMEM_SHARED((NP, D), jnp.float32),
        pltpu.VMEM((NCH, CHUNK), jnp.int32),
        pltpu.VMEM((4, CHUNK), jnp.int32),
        pltpu.VMEM((CHUNK, D), jnp.float32),
        pltpu.VMEM((CHUNK, D), jnp.float32),
        pltpu.SemaphoreType.DMA,
        pltpu.SemaphoreType.DMA,
        pltpu.SemaphoreType.DMA,
        pltpu.SemaphoreType.DMA,
        pltpu.SemaphoreType.DMA((4,)),
    ],
    name="gcn_edge_agg",
)
def _agg_feat(table, init0, init1, src_h, dst_h, out, acc, dst2d,
              sidx4, msg_a, msg_b, sg_a, sg_b, ss_a, ss_b, si4):
  core = jax.lax.axis_index("c")
  sub = jax.lax.axis_index("s")
  wid = core * NUM_SUBCORES + sub
  rbase = sub * ROWS_PER_SUB
  rows = pl.ds(rbase, ROWS_PER_SUB)
  ebase = wid * EPW

  @pl.when(core == 0)
  def _():
    pltpu.async_copy(init0.at[rows], acc.at[rows], ss_a)

  @pl.when(core == 1)
  def _():
    pltpu.async_copy(init1.at[rows], acc.at[rows], ss_a)

  pltpu.sync_copy(dst_h.at[wid], dst2d)

  def src_slice(c):
    return src_h.at[pl.ds(ebase + c * CHUNK, CHUNK)]

  def slot(c):
    return jax.lax.rem(c, 4) if not isinstance(c, int) else c % 4

  def start_idx(c):
    pltpu.async_copy(src_slice(c), sidx4.at[slot(c)], si4.at[slot(c)])

  def wait_idx(c):
    pltpu.make_async_copy(src_slice(c), sidx4.at[slot(c)],
                          si4.at[slot(c)]).wait()

  def start_gather(c, msg, sem):
    pltpu.async_copy(table.at[sidx4.at[slot(c)]], msg, sem)

  def wait_gather(c, msg, sem):
    pltpu.make_async_copy(table.at[sidx4.at[slot(c)]], msg, sem).wait()

  def start_scatter(c, msg, sem):
    pltpu.async_copy(msg, acc.at[dst2d.at[c]], sem, add=True)

  def wait_scatter(c, msg, sem):
    pltpu.make_async_copy(msg, acc.at[dst2d.at[c]], sem).wait()

  pltpu.sync_copy(src_slice(0), sidx4.at[0])
  pltpu.sync_copy(src_slice(1), sidx4.at[1])
  start_gather(0, msg_a, sg_a)
  start_gather(1, msg_b, sg_b)
  start_idx(2)
  start_idx(3)

  @pl.when(core == 0)
  def _():
    pltpu.make_async_copy(init0.at[rows], acc.at[rows], ss_a).wait()

  @pl.when(core == 1)
  def _():
    pltpu.make_async_copy(init1.at[rows], acc.at[rows], ss_a).wait()

  plsc.subcore_barrier()
  wait_gather(0, msg_a, sg_a)
  start_scatter(0, msg_a, ss_a)

  @pl.loop(1, NCH // 2 - 1)
  def _(j):
    c0 = 2 * j
    wait_scatter(c0 - 2, msg_a, ss_a)
    start_idx(c0 + 2)
    wait_idx(c0)
    start_gather(c0, msg_a, sg_a)
    wait_gather(c0 - 1, msg_b, sg_b)
    start_scatter(c0 - 1, msg_b, ss_b)
    wait_scatter(c0 - 1, msg_b, ss_b)
    start_idx(c0 + 3)
    wait_idx(c0 + 1)
    start_gather(c0 + 1, msg_b, sg_b)
    wait_gather(c0, msg_a, sg_a)
    start_scatter(c0, msg_a, ss_a)

  wait_scatter(NCH - 4, msg_a, ss_a)
  wait_idx(NCH - 2)
  start_gather(NCH - 2, msg_a, sg_a)
  wait_gather(NCH - 3, msg_b, sg_b)
  start_scatter(NCH - 3, msg_b, ss_b)
  wait_scatter(NCH - 3, msg_b, ss_b)
  wait_idx(NCH - 1)
  start_gather(NCH - 1, msg_b, sg_b)
  wait_gather(NCH - 2, msg_a, sg_a)
  start_scatter(NCH - 2, msg_a, ss_a)
  wait_gather(NCH - 1, msg_b, sg_b)
  start_scatter(NCH - 1, msg_b, ss_b)
  wait_scatter(NCH - 2, msg_a, ss_a)
  wait_scatter(NCH - 1, msg_b, ss_b)

  plsc.subcore_barrier()
  pltpu.sync_copy(acc.at[rows], out.at[core].at[rows])

_DEG_CHUNK = 2000

_sc_params = pltpu.CompilerParams()
if "needs_layout_passes" in pltpu.CompilerParams.__dataclass_fields__:
  _sc_params = dataclasses.replace(_sc_params, needs_layout_passes=False)


@functools.partial(
    pl.kernel,
    out_type=jax.ShapeDtypeStruct((NUM_WORKERS, NP), jnp.float32),
    mesh=plsc.VectorSubcoreMesh(core_axis_name="c", subcore_axis_name="s"),
    scratch_types=[
        pltpu.VMEM((NP,), jnp.float32),
        pltpu.VMEM((_DEG_CHUNK,), jnp.int32),
    ],
    compiler_params=_sc_params,
    name="gcn_degree",
)
def _deg_kernel(dst_h, out, hist, didx):
  core = jax.lax.axis_index("c")
  sub = jax.lax.axis_index("s")
  wid = core * NUM_SUBCORES + sub
  zeros16 = jnp.zeros((16,), jnp.float32)
  ones16 = jnp.ones((16,), jnp.float32)

  @pl.loop(0, NP // 16)
  def _(i):
    hist[pl.ds(i * 16, 16)] = zeros16

  ebase = wid * PER_WORKER

  @pl.loop(0, PER_WORKER // _DEG_CHUNK)
  def _(i):
    pltpu.sync_copy(dst_h.at[pl.ds(ebase + i * _DEG_CHUNK, _DEG_CHUNK)], didx)

    @pl.loop(0, _DEG_CHUNK // 16)
    def _(j):
      idx = didx[pl.ds(j * 16, 16)]
      plsc.addupdate_scatter(hist, [idx], ones16)

  pltpu.sync_copy(hist, out.at[wid])


def _p1a_body(x_ref, w1_ref, h_ref):
  h_ref[...] = jax.lax.dot_general(x_ref[...], w1_ref[...],
                                   (((1,), (1,)), ((), ())),
                                   preferred_element_type=jnp.float32)


def _p1_body(degp_ref, h_ref, y1_ref, dis_ref):
  ones_col = jnp.ones((NUM_WORKERS, 1), jnp.float32)
  deg = 1.0 + jax.lax.dot_general(degp_ref[...], ones_col,
                                  (((0,), (0,)), ((), ())),
                                  preferred_element_type=jnp.float32)
  dis = jax.lax.rsqrt(deg)
  y1_ref[...] = dis * h_ref[...]
  dis_ref[...] = dis


def _p2_body(p_ref, dis_ref, b1_ref, w2_ref, y2_ref):
  s = dis_ref[...] * (p_ref[0] + p_ref[1]) + b1_ref[...]
  h1r = jnp.maximum(s, 0.0)
  h2 = jax.lax.dot_general(h1r, w2_ref[...], (((1,), (1,)), ((), ())),
                           preferred_element_type=jnp.float32)
  y2_ref[...] = dis_ref[...] * h2


def _p3_body(q_ref, dis_ref, b2_ref, x_ref, o_ref):
  o_ref[...] = dis_ref[...] * (q_ref[0] + q_ref[1]) + b2_ref[...] + x_ref[...]


def kernel(x, edge_index, W1, b1, W2, b2):
  src = edge_index[0]
  dst = edge_index[1]
  xp = jnp.pad(x, ((0, NP - N), (0, 0)))
  zeros_feat = jnp.zeros((NP, D), jnp.float32)

  pad_rows = N + (jnp.arange(E_PAD - E, dtype=jnp.int32) % (NP - N))
  srcp = jnp.concatenate([src, pad_rows])
  dstp = jnp.concatenate([dst, pad_rows]).reshape(NUM_WORKERS, NCH, CHUNK)

  degp = _deg_kernel(dst)
  h1 = pl.pallas_call(
      _p1a_body,
      out_shape=jax.ShapeDtypeStruct((NP, D), jnp.float32),
  )(xp, W1)

  y1, dis = pl.pallas_call(
      _p1_body,
      out_shape=(jax.ShapeDtypeStruct((NP, D), jnp.float32),
                 jax.ShapeDtypeStruct((NP, 1), jnp.float32)),
  )(degp, h1)

  p = _agg_feat(y1, y1, zeros_feat, srcp, dstp)

  y2 = pl.pallas_call(
      _p2_body,
      out_shape=jax.ShapeDtypeStruct((NP, D), jnp.float32),
  )(p, dis, b1.reshape(1, D), W2)

  q = _agg_feat(y2, y2, zeros_feat, srcp, dstp)

  out = pl.pallas_call(
      _p3_body,
      out_shape=jax.ShapeDtypeStruct((NP, D), jnp.float32),
  )(q, dis, b2.reshape(1, D), xp)

  return out[:N]
